# baseline (speedup 1.0000x reference)
import functools

import jax
import jax.numpy as jnp
from jax import lax
from jax.experimental import pallas as pl
from jax.experimental.pallas import tpu as pltpu
from jax.experimental.pallas import tpu_sc as plsc

N = 10000
M = 32
F = 42
FP = 48
H1, H2 = 64, 32
NPAD = 10016
NW = 32
AP = 313
BLK = 1024
DF = M * F * 3
DFB = 4064


def _mlp_body(img_ref, w1_ref, b1_ref, w2_ref, b2_ref, w3t_ref, b3_ref,
              ei_ref, de_ref, etot_ref):
    i = pl.program_id(0)
    img = img_ref[...]
    z1 = jnp.dot(img, w1_ref[...],
                 preferred_element_type=jnp.float32) + b1_ref[...]
    a1 = jax.nn.sigmoid(z1)
    z2 = jnp.dot(a1, w2_ref[...],
                 preferred_element_type=jnp.float32) + b2_ref[...]
    a2 = jax.nn.sigmoid(z2)
    w3t = w3t_ref[...]
    ei = jnp.sum(a2 * w3t, axis=1, keepdims=True) + b3_ref[...]
    rid = i * BLK + lax.broadcasted_iota(jnp.int32, (BLK, 1), 0)
    valid = rid < N
    ei = jnp.where(valid, ei, 0.0)
    g2 = a2 * (1.0 - a2) * w3t
    g1 = lax.dot_general(g2, w2_ref[...], (((1,), (1,)), ((), ())),
                         preferred_element_type=jnp.float32)
    g1 = g1 * (a1 * (1.0 - a1))
    de = lax.dot_general(g1, w1_ref[...], (((1,), (1,)), ((), ())),
                         preferred_element_type=jnp.float32)
    de = jnp.where(valid, de, 0.0)
    ei_ref[...] = ei
    de_ref[...] = de

    @pl.when(i == 0)
    def _init():
        etot_ref[...] = jnp.zeros_like(etot_ref)

    etot_ref[...] += jnp.sum(ei)


def _run_mlp(img_pad, w1p, b1r, w2, b2r, w3t, b3r):
    grid = (NPAD + BLK - 1) // BLK
    full = lambda s: pl.BlockSpec(s, lambda i: tuple(0 for _ in s))
    return pl.pallas_call(
        _mlp_body,
        grid=(grid,),
        in_specs=[
            pl.BlockSpec((BLK, FP), lambda i: (i, 0)),
            full((FP, H1)), full((1, H1)),
            full((H1, H2)), full((1, H2)),
            full((1, H2)), full((1, 1)),
        ],
        out_specs=[
            pl.BlockSpec((BLK, 1), lambda i: (i, 0)),
            pl.BlockSpec((BLK, FP), lambda i: (i, 0)),
            pl.BlockSpec((1, 1), lambda i: (0, 0)),
        ],
        out_shape=[
            jax.ShapeDtypeStruct((NPAD, 1), jnp.float32),
            jax.ShapeDtypeStruct((NPAD, FP), jnp.float32),
            jax.ShapeDtypeStruct((1, 1), jnp.float32),
        ],
    )(img_pad, w1p, b1r, w2, b2r, w3t, b3r)


def _force_body(de_hbm, nbr_hbm, dfeat_hbm, out_hbm,
                idx_slab, df_ring, deg0, deg1, out_ring,
                sem_df0, sem_df1, sem_dg0, sem_dg1, sem_out):
    nc = 2
    wid = lax.axis_index("s") * nc + lax.axis_index("c")
    start = wid * AP
    na = jnp.minimum(N - start, AP)

    sems_df = (sem_df0, sem_df1)
    sems_dg = (sem_dg0, sem_dg1)
    degs = (deg0, deg1)

    zero16 = jnp.zeros((16,), jnp.float32)
    for slot in range(2):
        for t in range(DF, DFB, 16):
            df_ring[pl.ds(slot * DFB + t, 16)] = zero16

    pltpu.sync_copy(nbr_hbm.at[pl.ds(start * M, AP * M)], idx_slab)

    def _remap(t, _):
        v = idx_slab[pl.ds(t * 16, 16)]
        idx_slab[pl.ds(t * 16, 16)] = jnp.where(v > 0, v - 1, N)
        return 0

    lax.fori_loop(0, AP * M // 16, _remap, 0)

    i3 = lax.iota(jnp.int32, 16) * 3
    io16 = lax.iota(jnp.int32, 16)

    def _fire(a, slot):
        atom = start + a
        pltpu.async_copy(dfeat_hbm.at[pl.ds(atom * DF, DF)],
                         df_ring.at[pl.ds(slot * DFB, DF)], sems_df[slot])
        pltpu.async_copy(de_hbm.at[idx_slab.at[pl.ds(a * M, M)]],
                         degs[slot], sems_dg[slot])

    def _wait(slot):
        pltpu.make_async_copy(dfeat_hbm.at[pl.ds(0, DF)],
                              df_ring.at[pl.ds(slot * DFB, DF)],
                              sems_df[slot]).wait()
        pltpu.make_async_copy(de_hbm.at[idx_slab.at[pl.ds(0, M)]],
                              degs[slot], sems_dg[slot]).wait()

    def _compute(a, slot):
        atom = start + a
        deg = degs[slot]
        dfoff = slot * DFB

        def _m_body(m, accs):
            a0, a1, a2 = accs
            dbase = dfoff + m * (F * 3)
            for c in range(3):
                dev = deg[m, pl.ds(16 * c, 16)]
                g0 = plsc.load_gather(df_ring, [i3 + (dbase + 48 * c)])
                g1 = plsc.load_gather(df_ring, [i3 + (dbase + 48 * c + 1)])
                g2 = plsc.load_gather(df_ring, [i3 + (dbase + 48 * c + 2)])
                a0 = a0 + dev * g0
                a1 = a1 + dev * g1
                a2 = a2 + dev * g2
            return (a0, a1, a2)

        z = jnp.zeros((16,), jnp.float32)
        acc0, acc1, acc2 = lax.fori_loop(0, M, _m_body, (z, z, z))
        f0 = jnp.sum(acc0)
        f1 = jnp.sum(acc1)
        f2 = jnp.sum(acc2)
        fv = jnp.where(io16 == 0, f0,
                       jnp.where(io16 == 1, f1,
                                 jnp.where(io16 == 2, f2, 0.0)))
        oslot = lax.rem(a, 4)

        @pl.when(a >= 4)
        def _drain_one():
            pltpu.make_async_copy(out_ring.at[pl.ds(0, 16)],
                                  out_hbm.at[pl.ds(0, 16)], sem_out).wait()

        out_ring[pl.ds(oslot * 16, 16)] = fv
        pltpu.async_copy(out_ring.at[pl.ds(oslot * 16, 16)],
                         out_hbm.at[pl.ds(atom * 16, 16)], sem_out)

    _fire(0, 0)
    npairs = (AP + 1) // 2

    def _pair(g, _):
        for b in range(2):
            a = 2 * g + b
            slot = b

            @pl.when(a + 1 < na)
            def _f():
                _fire(a + 1, 1 - slot)

            @pl.when(a < na)
            def _c():
                _wait(slot)
                _compute(a, slot)

        return 0

    lax.fori_loop(0, npairs, _pair, 0)

    for w in range(4):
        pltpu.make_async_copy(out_ring.at[pl.ds(0, 16)],
                              out_hbm.at[pl.ds(0, 16)], sem_out).wait()


def _make_force_call():
  return functools.partial(
    pl.kernel,
    out_type=jax.ShapeDtypeStruct((NPAD * 16,), jnp.float32),
    mesh=plsc.VectorSubcoreMesh(core_axis_name="c", subcore_axis_name="s",
                                num_cores=2, num_subcores=16),
    compiler_params=pltpu.CompilerParams(needs_layout_passes=False,
                                         use_tc_tiling_on_sc=False),
    scratch_types=[
        pltpu.VMEM((AP * M,), jnp.int32),
        pltpu.VMEM((2 * DFB,), jnp.float32),
        pltpu.VMEM((M, FP), jnp.float32),
        pltpu.VMEM((M, FP), jnp.float32),
        pltpu.VMEM((4 * 16,), jnp.float32),
        pltpu.SemaphoreType.DMA,
        pltpu.SemaphoreType.DMA,
        pltpu.SemaphoreType.DMA,
        pltpu.SemaphoreType.DMA,
        pltpu.SemaphoreType.DMA,
    ],
  )(_force_body)


def kernel(image, dfeat, neighbor, Egroup_weight, divider, W1, b1, W2, b2, W3, b3):
    del Egroup_weight, divider
    img = image[0]
    img_pad = jnp.pad(img, ((0, NPAD - N), (0, FP - F)))
    w1p = jnp.pad(W1, ((0, FP - F), (0, 0)))
    b1r = b1.reshape(1, H1)
    b2r = b2.reshape(1, H2)
    w3t = W3.reshape(1, H2)
    b3r = b3.reshape(1, 1)

    ei_pad, de_pad, etot = _run_mlp(img_pad, w1p, b1r, W2, b2r, w3t, b3r)

    nbr_flat = jnp.pad(neighbor[0], ((0, NPAD - N), (0, 0))).reshape(-1)
    nbr_flat = nbr_flat.astype(jnp.int32)
    dfeat_flat = dfeat.reshape(N * DF)

    force_pad = _make_force_call()(de_pad, nbr_flat, dfeat_flat)

    Ei = ei_pad[:N, 0][None]
    Etot = etot
    Force = force_pad.reshape(NPAD, 16)[:N, :3][None]
    return (Etot, Ei, Force)


# speedup vs baseline: 17.5637x; 17.5637x over previous
import functools

import jax
import jax.numpy as jnp
from jax import lax
from jax.experimental import pallas as pl
from jax.experimental.pallas import tpu as pltpu
from jax.experimental.pallas import tpu_sc as plsc

N = 10000
M = 32
F = 42
FP = 48
D3 = 3
FD = F * D3
H1, H2 = 64, 32
NPAD = 10016
BLK = 1024
B2 = N * M
NWK = 32
PW = B2 // NWK
CH = 1000
NCH = PW // CH
BA = 400
BAM = BA * M


def _mlp_body(img_ref, w1_ref, b1_ref, w2_ref, b2_ref, w3t_ref, b3_ref,
              ei_ref, de_ref, etot_ref):
    i = pl.program_id(0)
    img = img_ref[...]
    z1 = jnp.dot(img, w1_ref[...],
                 preferred_element_type=jnp.float32) + b1_ref[...]
    a1 = jax.nn.sigmoid(z1)
    z2 = jnp.dot(a1, w2_ref[...],
                 preferred_element_type=jnp.float32) + b2_ref[...]
    a2 = jax.nn.sigmoid(z2)
    w3t = w3t_ref[...]
    ei = jnp.sum(a2 * w3t, axis=1, keepdims=True) + b3_ref[...]
    rid = i * BLK + lax.broadcasted_iota(jnp.int32, (BLK, 1), 0)
    valid = rid < N
    ei = jnp.where(valid, ei, 0.0)
    g2 = a2 * (1.0 - a2) * w3t
    g1 = lax.dot_general(g2, w2_ref[...], (((1,), (1,)), ((), ())),
                         preferred_element_type=jnp.float32)
    g1 = g1 * (a1 * (1.0 - a1))
    de = lax.dot_general(g1, w1_ref[...], (((1,), (1,)), ((), ())),
                         preferred_element_type=jnp.float32)
    de = jnp.where(valid, de, 0.0)
    ei_ref[...] = ei
    de_ref[...] = de

    @pl.when(i == 0)
    def _init():
        etot_ref[...] = jnp.zeros_like(etot_ref)

    etot_ref[...] += jnp.sum(ei)


def _run_mlp(img_pad, w1p, b1r, w2, b2r, w3t, b3r):
    grid = (NPAD + BLK - 1) // BLK
    full = lambda s: pl.BlockSpec(s, lambda i: tuple(0 for _ in s))
    return pl.pallas_call(
        _mlp_body,
        grid=(grid,),
        in_specs=[
            pl.BlockSpec((BLK, FP), lambda i: (i, 0)),
            full((FP, H1)), full((1, H1)),
            full((H1, H2)), full((1, H2)),
            full((1, H2)), full((1, 1)),
        ],
        out_specs=[
            pl.BlockSpec((BLK, 1), lambda i: (i, 0)),
            pl.BlockSpec((BLK, FP), lambda i: (i, 0)),
            pl.BlockSpec((1, 1), lambda i: (0, 0)),
        ],
        out_shape=[
            jax.ShapeDtypeStruct((NPAD, 1), jnp.float32),
            jax.ShapeDtypeStruct((NPAD, FP), jnp.float32),
            jax.ShapeDtypeStruct((1, 1), jnp.float32),
        ],
    )(img_pad, w1p, b1r, w2, b2r, w3t, b3r)


def _gather_body(de_hbm, idx_hbm, out_hbm,
                 idx0, idx1, rows0, rows1, gs0, gs1, os0, os1):
    nc = 2
    wid = lax.axis_index("s") * nc + lax.axis_index("c")
    base = wid * PW
    idxs = (idx0, idx1)
    rows = (rows0, rows1)
    gss = (gs0, gs1)
    oss = (os0, os1)

    def fire(k, s):
        pltpu.sync_copy(idx_hbm.at[pl.ds(base + k * CH, CH)], idxs[s])
        pltpu.async_copy(de_hbm.at[idxs[s]], rows[s], gss[s])

    def drain(j, s):
        pltpu.make_async_copy(de_hbm.at[idxs[s]], rows[s], gss[s]).wait()
        pltpu.async_copy(rows[s], out_hbm.at[pl.ds(base + j * CH, CH)], oss[s])

    for k in range(NCH + 1):
        s = k % 2
        if k >= 2:
            pltpu.make_async_copy(rows[s], out_hbm.at[pl.ds(base, CH)],
                                  oss[s]).wait()
        if k < NCH:
            fire(k, s)
        if k >= 1:
            drain(k - 1, (k - 1) % 2)

    s = (NCH - 1) % 2
    pltpu.make_async_copy(rows[s], out_hbm.at[pl.ds(base, CH)], oss[s]).wait()


def _run_gather(de_pad, idx2):
    call = functools.partial(
        pl.kernel,
        out_type=jax.ShapeDtypeStruct((B2, FP), jnp.float32),
        mesh=plsc.VectorSubcoreMesh(core_axis_name="c", subcore_axis_name="s",
                                    num_cores=2, num_subcores=16),
        compiler_params=pltpu.CompilerParams(needs_layout_passes=False,
                                             use_tc_tiling_on_sc=False),
        scratch_types=[
            pltpu.VMEM((CH,), jnp.int32),
            pltpu.VMEM((CH,), jnp.int32),
            pltpu.VMEM((CH, FP), jnp.float32),
            pltpu.VMEM((CH, FP), jnp.float32),
            pltpu.SemaphoreType.DMA,
            pltpu.SemaphoreType.DMA,
            pltpu.SemaphoreType.DMA,
            pltpu.SemaphoreType.DMA,
        ],
    )
    return call(_gather_body)(de_pad, idx2)


def _contract_body(a_ref, d_ref, out_ref):
    a = a_ref[...]
    d = d_ref[...]
    fidx = lax.broadcasted_iota(jnp.int32, (FP, FD), 0)
    cidx = lax.broadcasted_iota(jnp.int32, (FP, FD), 1)
    expand = (cidx // D3 == fidx).astype(jnp.float32)
    e1 = jnp.dot(a, expand, preferred_element_type=jnp.float32)
    p = e1 * d
    ps = jnp.sum(p.reshape(BA, M, FD), axis=1)
    ridx = lax.broadcasted_iota(jnp.int32, (FD, 8), 0)
    didx = lax.broadcasted_iota(jnp.int32, (FD, 8), 1)
    sel = (ridx % D3 == didx).astype(jnp.float32)
    out_ref[...] = jnp.dot(ps, sel, preferred_element_type=jnp.float32)


def _run_contract(a_rows, dflat):
    grid = N // BA
    return pl.pallas_call(
        _contract_body,
        grid=(grid,),
        in_specs=[
            pl.BlockSpec((BAM, FP), lambda i: (i, 0)),
            pl.BlockSpec((BAM, FD), lambda i: (i, 0)),
        ],
        out_specs=pl.BlockSpec((BA, 8), lambda i: (i, 0)),
        out_shape=jax.ShapeDtypeStruct((N, 8), jnp.float32),
    )(a_rows, dflat)


def kernel(image, dfeat, neighbor, Egroup_weight, divider, W1, b1, W2, b2, W3, b3):
    del Egroup_weight, divider
    img = image[0]
    img_pad = jnp.pad(img, ((0, NPAD - N), (0, FP - F)))
    w1p = jnp.pad(W1, ((0, FP - F), (0, 0)))
    b1r = b1.reshape(1, H1)
    b2r = b2.reshape(1, H2)
    w3t = W3.reshape(1, H2)
    b3r = b3.reshape(1, 1)

    ei_pad, de_pad, etot = _run_mlp(img_pad, w1p, b1r, W2, b2r, w3t, b3r)

    nbr = neighbor[0].astype(jnp.int32).reshape(B2)
    idx2 = jnp.where(nbr > 0, nbr - 1, N)

    a_rows = _run_gather(de_pad, idx2)

    dflat = dfeat.reshape(B2, FD)
    force8 = _run_contract(a_rows, dflat)

    Ei = ei_pad[:N, 0][None]
    Etot = etot
    Force = force8[:, :3][None]
    return (Etot, Ei, Force)
